# baseline (speedup 1.0000x reference)
import jax
import jax.numpy as jnp
from jax import lax
from jax.experimental import pallas as pl
from jax.experimental.pallas import tpu as pltpu
from jax.experimental.pallas import tpu_sc as plsc

_NW = 32
_ROW = 32768
_B = 4
_PROWS = 256
_CHUNKS = _PROWS // _NW
_T = _CHUNKS * _B
_U = 4


def _sc_body(x_hbm, pos_hbm, out_hbm, pos_v, x_v, spos, sin0, sin1, sout0, sout1):
    wid = lax.axis_index("s") * 2 + lax.axis_index("c")
    p0 = wid * _CHUNKS
    sin = (sin0, sin1)
    sout = (sout0, sout1)

    def in_copy(t):
        chunk, b = divmod(t, _B)
        row = b * _PROWS + p0 + chunk
        return pltpu.make_async_copy(x_hbm.at[row], x_v.at[t % 2], sin[t % 2])

    def out_copy(t):
        chunk, b = divmod(t, _B)
        row = b * _PROWS + p0 + chunk
        return pltpu.make_async_copy(x_v.at[t % 2], out_hbm.at[row], sout[t % 2])

    def pos_copy(chunk):
        return pltpu.make_async_copy(pos_hbm.at[p0 + chunk], pos_v, spos)

    def compute(t):
        bf = t % 2

        def body(j, _):
            o = j * (_U * 16)
            for u in range(_U):
                off = o + u * 16
                x_v[bf, pl.ds(off, 16)] = (
                    x_v[bf, pl.ds(off, 16)] + pos_v[pl.ds(off, 16)]
                )
            return 0

        lax.fori_loop(0, _ROW // (_U * 16), body, 0)

    pos_copy(0).start()
    in_copy(0).start()
    in_copy(1).start()
    for t in range(_T):
        chunk, b = divmod(t, _B)
        if t >= 2:
            out_copy(t - 2).wait()
        if t + 2 < _T:
            in_copy(t + 2).start()
        in_copy(t).wait()
        if b == 0:
            pos_copy(chunk).wait()
        compute(t)
        out_copy(t).start()
        if b == _B - 1 and chunk + 1 < _CHUNKS:
            pos_copy(chunk + 1).start()
    out_copy(_T - 2).wait()
    out_copy(_T - 1).wait()


def kernel(x, pos_table):
    B, S, D = x.shape
    x2 = x.reshape(B * S * D // _ROW, _ROW)
    pos2 = pos_table.reshape(pos_table.size // _ROW, _ROW)
    mesh = plsc.VectorSubcoreMesh(core_axis_name="c", subcore_axis_name="s")
    out = pl.kernel(
        _sc_body,
        mesh=mesh,
        out_type=jax.ShapeDtypeStruct(x2.shape, x.dtype),
        scratch_types=[
            pltpu.VMEM((_ROW,), jnp.float32),
            pltpu.VMEM((2, _ROW), jnp.float32),
            pltpu.SemaphoreType.DMA,
            pltpu.SemaphoreType.DMA,
            pltpu.SemaphoreType.DMA,
            pltpu.SemaphoreType.DMA,
            pltpu.SemaphoreType.DMA,
        ],
    )(x2, pos2)
    return out.reshape(B, S, D)


# speedup vs baseline: 1.0021x; 1.0021x over previous
import jax
import jax.numpy as jnp
from jax import lax
from jax.experimental import pallas as pl
from jax.experimental.pallas import tpu as pltpu
from jax.experimental.pallas import tpu_sc as plsc

_NW = 32
_ROW = 16384
_B = 4
_PROWS = 512
_CHUNKS = _PROWS // _NW
_T = _CHUNKS * _B
_U = 4
_NB = 3


def _sc_body(x_hbm, pos_hbm, out_hbm, pos_v, x_v,
             spos0, spos1, sin0, sin1, sin2, sout0, sout1, sout2):
    wid = lax.axis_index("s") * 2 + lax.axis_index("c")
    p0 = wid * _CHUNKS
    spos = (spos0, spos1)
    sin = (sin0, sin1, sin2)
    sout = (sout0, sout1, sout2)

    def in_copy(t):
        chunk, b = divmod(t, _B)
        row = b * _PROWS + p0 + chunk
        return pltpu.make_async_copy(x_hbm.at[pl.ds(row, 1)], x_v.at[pl.ds(t % _NB, 1)], sin[t % _NB])

    def out_copy(t):
        chunk, b = divmod(t, _B)
        row = b * _PROWS + p0 + chunk
        return pltpu.make_async_copy(x_v.at[pl.ds(t % _NB, 1)], out_hbm.at[pl.ds(row, 1)], sout[t % _NB])

    def pos_copy(chunk):
        return pltpu.make_async_copy(
            pos_hbm.at[pl.ds(p0 + chunk, 1)], pos_v.at[pl.ds(chunk % 2, 1)], spos[chunk % 2])

    def compute(t):
        bf = t % _NB
        pb = (t // _B) % 2

        def body(j, _):
            o = j * (_U * 16)
            for u in range(_U):
                off = o + u * 16
                x_v[bf, pl.ds(off, 16)] = (
                    x_v[bf, pl.ds(off, 16)] + pos_v[pb, pl.ds(off, 16)]
                )
            return 0

        lax.fori_loop(0, _ROW // (_U * 16), body, 0)

    pos_copy(0).start()
    pos_copy(1).start()
    in_copy(0).start()
    in_copy(1).start()
    for t in range(_T):
        chunk, b = divmod(t, _B)
        in_copy(t).wait()
        if b == 0:
            pos_copy(chunk).wait()
        compute(t)
        out_copy(t).start()
        if b == _B - 1 and chunk + 2 < _CHUNKS:
            pos_copy(chunk + 2).start()
        if t >= 1:
            out_copy(t - 1).wait()
        if t + 2 < _T:
            in_copy(t + 2).start()
    out_copy(_T - 1).wait()


def kernel(x, pos_table):
    B, S, D = x.shape
    x2 = x.reshape(B * S * D // _ROW, _ROW)
    pos2 = pos_table.reshape(pos_table.size // _ROW, _ROW)
    mesh = plsc.VectorSubcoreMesh(core_axis_name="c", subcore_axis_name="s")
    out = pl.kernel(
        _sc_body,
        mesh=mesh,
        out_type=jax.ShapeDtypeStruct(x2.shape, x.dtype),
        scratch_types=[
            pltpu.VMEM((2, _ROW), jnp.float32),
            pltpu.VMEM((_NB, _ROW), jnp.float32),
            pltpu.SemaphoreType.DMA,
            pltpu.SemaphoreType.DMA,
            pltpu.SemaphoreType.DMA,
            pltpu.SemaphoreType.DMA,
            pltpu.SemaphoreType.DMA,
            pltpu.SemaphoreType.DMA,
            pltpu.SemaphoreType.DMA,
            pltpu.SemaphoreType.DMA,
        ],
    )(x2, pos2)
    return out.reshape(B, S, D)


# speedup vs baseline: 6.0675x; 6.0548x over previous
import jax
import jax.numpy as jnp
from jax import lax
from jax.experimental import pallas as pl
from jax.experimental.pallas import tpu as pltpu
from jax.experimental.pallas import tpu_sc as plsc

_NW = 32
_R = 8
_B = 4
_S = 4096
_D = 2048
_CHUNKS = _S // _NW // _R
_T = _CHUNKS * _B
_VECS = _R * (_D // 16)


def _sc_body(x_hbm, pos_hbm, out_hbm,
             p0_v, p1_v, x0_v, x1_v, x2_v,
             spos0, spos1, sin0, sin1, sin2, sout0, sout1, sout2):
    wid = lax.axis_index("s") * 2 + lax.axis_index("c")
    s0 = wid * (_S // _NW)
    pbufs = (p0_v, p1_v)
    xbufs = (x0_v, x1_v, x2_v)
    spos = (spos0, spos1)
    sin = (sin0, sin1, sin2)
    sout = (sout0, sout1, sout2)

    def in_copy(t):
        chunk, b = divmod(t, _B)
        row = b * _S + s0 + chunk * _R
        return pltpu.make_async_copy(
            x_hbm.at[pl.ds(row, _R)], xbufs[t % 3], sin[t % 3])

    def out_copy(t):
        chunk, b = divmod(t, _B)
        row = b * _S + s0 + chunk * _R
        return pltpu.make_async_copy(
            xbufs[t % 3], out_hbm.at[pl.ds(row, _R)], sout[t % 3])

    def pos_copy(chunk):
        return pltpu.make_async_copy(
            pos_hbm.at[pl.ds(s0 + chunk * _R, _R)], pbufs[chunk % 2],
            spos[chunk % 2])

    def compute(t):
        xb = xbufs[t % 3]
        pb = pbufs[(t // _B) % 2]

        @plsc.parallel_loop(0, _VECS, unroll=8)
        def body(i):
            r = i // (_D // 16)
            c = (i - r * (_D // 16)) * 16
            xb[r, pl.ds(c, 16)] = xb[r, pl.ds(c, 16)] + pb[r, pl.ds(c, 16)]

    pos_copy(0).start()
    pos_copy(1).start()
    in_copy(0).start()
    in_copy(1).start()
    for t in range(_T):
        chunk, b = divmod(t, _B)
        in_copy(t).wait()
        if b == 0:
            pos_copy(chunk).wait()
        compute(t)
        out_copy(t).start()
        if b == _B - 1 and chunk + 2 < _CHUNKS:
            pos_copy(chunk + 2).start()
        if t >= 1:
            out_copy(t - 1).wait()
        if t + 2 < _T:
            in_copy(t + 2).start()
    out_copy(_T - 1).wait()


def kernel(x, pos_table):
    B, S, D = x.shape
    x2 = x.reshape(B * S, D)
    mesh = plsc.VectorSubcoreMesh(core_axis_name="c", subcore_axis_name="s")
    out = pl.kernel(
        _sc_body,
        mesh=mesh,
        out_type=jax.ShapeDtypeStruct((B * S, D), x.dtype),
        scratch_types=[
            pltpu.VMEM((_R, _D), jnp.float32),
            pltpu.VMEM((_R, _D), jnp.float32),
            pltpu.VMEM((_R, _D), jnp.float32),
            pltpu.VMEM((_R, _D), jnp.float32),
            pltpu.VMEM((_R, _D), jnp.float32),
            pltpu.SemaphoreType.DMA,
            pltpu.SemaphoreType.DMA,
            pltpu.SemaphoreType.DMA,
            pltpu.SemaphoreType.DMA,
            pltpu.SemaphoreType.DMA,
            pltpu.SemaphoreType.DMA,
            pltpu.SemaphoreType.DMA,
            pltpu.SemaphoreType.DMA,
        ],
    )(x2, pos_table)
    return out.reshape(B, S, D)
